# baseline (device time: 114040 ns/iter reference)
import jax
import jax.numpy as jnp
from jax import lax
from jax.experimental import pallas as pl
from jax.experimental.pallas import tpu as pltpu

N_DEV = 8
B, S, H, Dh, Dr = 2, 256, 16, 64, 32
D = 1024
DC = 64
BS = B * S


def _dot(a, b):
    return jnp.dot(a, b, preferred_element_type=jnp.float32)


def kernel(x, Wdkv, Wuk, Wuv, Wq, Wqr, Wkr, Wo):
    def body(x_ref, wdkv_ref, wuk_ref, wuv_ref, wq_ref, wqr_ref, wkr_ref,
             wo_ref, out_ref,
             c_mine, w_mine, c_buf, w_buf,
             c_send_sems, c_recv_sems, w_send_sems, w_recv_sems):
        my = lax.axis_index("i")
        left = (my - 1) % N_DEV
        right = (my + 1) % N_DEV

        barrier = pltpu.get_barrier_semaphore()
        for nbr in (left, right):
            pl.semaphore_signal(barrier, inc=1, device_id=(nbr,),
                                device_id_type=pl.DeviceIdType.MESH)
        pl.semaphore_wait(barrier, 2)

        for b in range(B):
            c_mine[b * S:(b + 1) * S, :] = _dot(x_ref[b], wdkv_ref[...])
        w_mine[:, :D] = wuk_ref[...]
        w_mine[:, D:] = wuv_ref[...]

        K = _dot(c_mine[...], wuk_ref[...])
        V = _dot(c_mine[...], wuv_ref[...])

        for h in range(N_DEV - 1):
            c_src = c_mine if h == 0 else c_buf.at[h - 1]
            w_src = w_mine if h == 0 else w_buf.at[h - 1]
            c_rdma = pltpu.make_async_remote_copy(
                src_ref=c_src, dst_ref=c_buf.at[h],
                send_sem=c_send_sems.at[h], recv_sem=c_recv_sems.at[h],
                device_id=(right,), device_id_type=pl.DeviceIdType.MESH)
            w_rdma = pltpu.make_async_remote_copy(
                src_ref=w_src, dst_ref=w_buf.at[h],
                send_sem=w_send_sems.at[h], recv_sem=w_recv_sems.at[h],
                device_id=(right,), device_id_type=pl.DeviceIdType.MESH)
            c_rdma.start()
            w_rdma.start()
            c_rdma.wait()
            w_rdma.wait()
            K = K + _dot(c_buf[h], w_buf[h, :, :D])
            V = V + _dot(c_buf[h], w_buf[h, :, D:])

        scale = (Dh + Dr) ** -0.5
        for b in range(B):
            xb = x_ref[b]
            Qb = _dot(xb, wq_ref[...])
            Qrb = _dot(xb, wqr_ref[...])
            Krb = _dot(xb, wkr_ref[...])
            Kb = K[b * S:(b + 1) * S, :]
            Vb = V[b * S:(b + 1) * S, :]
            o_cols = []
            for h in range(H):
                Qh = Qb[:, h * Dh:(h + 1) * Dh]
                Kh = Kb[:, h * Dh:(h + 1) * Dh]
                Qrh = Qrb[:, h * Dr:(h + 1) * Dr]
                s = (_dot(Qh, Kh.T) + _dot(Qrh, Krb.T)) * scale
                m = jnp.max(s, axis=-1, keepdims=True)
                e = jnp.exp(s - m)
                p = e / jnp.sum(e, axis=-1, keepdims=True)
                o_cols.append(_dot(p, Vb[:, h * Dh:(h + 1) * Dh]))
            Ob = jnp.concatenate(o_cols, axis=1)
            out_ref[b, :, :] = _dot(Ob, wo_ref[...])

    f32 = jnp.float32
    return pl.pallas_call(
        body,
        out_shape=jax.ShapeDtypeStruct((B, S, D), f32),
        in_specs=[pl.BlockSpec(memory_space=pltpu.VMEM)] * 8,
        out_specs=pl.BlockSpec(memory_space=pltpu.VMEM),
        scratch_shapes=[
            pltpu.VMEM((BS, DC), f32),
            pltpu.VMEM((DC, 2 * D), f32),
            pltpu.VMEM((N_DEV - 1, BS, DC), f32),
            pltpu.VMEM((N_DEV - 1, DC, 2 * D), f32),
            pltpu.SemaphoreType.DMA((N_DEV - 1,)),
            pltpu.SemaphoreType.DMA((N_DEV - 1,)),
            pltpu.SemaphoreType.DMA((N_DEV - 1,)),
            pltpu.SemaphoreType.DMA((N_DEV - 1,)),
        ],
        compiler_params=pltpu.CompilerParams(collective_id=0),
    )(x, Wdkv, Wuk, Wuv, Wq, Wqr, Wkr, Wo)


# device time: 91039 ns/iter; 1.2526x vs baseline; 1.2526x over previous
import jax
import jax.numpy as jnp
from jax import lax
from jax.experimental import pallas as pl
from jax.experimental.pallas import tpu as pltpu

N_DEV = 8
B, S, H, Dh, Dr = 2, 256, 16, 64, 32
D = 1024
DC = 64
BS = B * S


def _dot(a, b):
    return jnp.dot(a, b, preferred_element_type=jnp.float32)


def kernel(x, Wdkv, Wuk, Wuv, Wq, Wqr, Wkr, Wo):
    def body(x_ref, wdkv_ref, wuk_ref, wuv_ref, wq_ref, wqr_ref, wkr_ref,
             wo_ref, out_ref,
             c_mine, w_mine, c_buf, w_buf,
             c_send_sems, c_recv_sems, w_send_sems, w_recv_sems):
        my = lax.axis_index("i")

        barrier = pltpu.get_barrier_semaphore()
        for k in range(1, N_DEV):
            pl.semaphore_signal(barrier, inc=1, device_id=((my + k) % N_DEV,),
                                device_id_type=pl.DeviceIdType.MESH)
        pl.semaphore_wait(barrier, N_DEV - 1)

        for b in range(B):
            c_mine[b * S:(b + 1) * S, :] = _dot(x_ref[b], wdkv_ref[...])
        w_mine[:, :D] = wuk_ref[...]
        w_mine[:, D:] = wuv_ref[...]

        sends = []
        for k in range(1, N_DEV):
            peer = (my + k) % N_DEV
            slot = N_DEV - 1 - k
            c_rdma = pltpu.make_async_remote_copy(
                src_ref=c_mine, dst_ref=c_buf.at[slot],
                send_sem=c_send_sems.at[k - 1], recv_sem=c_recv_sems.at[slot],
                device_id=(peer,), device_id_type=pl.DeviceIdType.MESH)
            w_rdma = pltpu.make_async_remote_copy(
                src_ref=w_mine, dst_ref=w_buf.at[slot],
                send_sem=w_send_sems.at[k - 1], recv_sem=w_recv_sems.at[slot],
                device_id=(peer,), device_id_type=pl.DeviceIdType.MESH)
            c_rdma.start()
            w_rdma.start()
            sends.append((c_rdma, w_rdma))

        K = _dot(c_mine[...], wuk_ref[...])
        V = _dot(c_mine[...], wuv_ref[...])
        Qs, Qrs, Krs = [], [], []
        for b in range(B):
            xb = x_ref[b]
            Qs.append(_dot(xb, wq_ref[...]))
            Qrs.append(_dot(xb, wqr_ref[...]))
            Krs.append(_dot(xb, wkr_ref[...]))

        for j in range(N_DEV - 1):
            recv_c = pltpu.make_async_remote_copy(
                src_ref=c_buf.at[j], dst_ref=c_buf.at[j],
                send_sem=c_send_sems.at[j], recv_sem=c_recv_sems.at[j],
                device_id=(my,), device_id_type=pl.DeviceIdType.MESH)
            recv_w = pltpu.make_async_remote_copy(
                src_ref=w_buf.at[j], dst_ref=w_buf.at[j],
                send_sem=w_send_sems.at[j], recv_sem=w_recv_sems.at[j],
                device_id=(my,), device_id_type=pl.DeviceIdType.MESH)
            recv_c.wait_recv()
            recv_w.wait_recv()
            K = K + _dot(c_buf[j], w_buf[j, :, :D])
            V = V + _dot(c_buf[j], w_buf[j, :, D:])

        scale = (Dh + Dr) ** -0.5
        for b in range(B):
            Qb, Qrb, Krb = Qs[b], Qrs[b], Krs[b]
            Kb = K[b * S:(b + 1) * S, :]
            Vb = V[b * S:(b + 1) * S, :]
            o_cols = []
            for h in range(H):
                Qh = Qb[:, h * Dh:(h + 1) * Dh]
                Kh = Kb[:, h * Dh:(h + 1) * Dh]
                Qrh = Qrb[:, h * Dr:(h + 1) * Dr]
                s = (_dot(Qh, Kh.T) + _dot(Qrh, Krb.T)) * scale
                m = jnp.max(s, axis=-1, keepdims=True)
                e = jnp.exp(s - m)
                p = e / jnp.sum(e, axis=-1, keepdims=True)
                o_cols.append(_dot(p, Vb[:, h * Dh:(h + 1) * Dh]))
            Ob = jnp.concatenate(o_cols, axis=1)
            out_ref[b, :, :] = _dot(Ob, wo_ref[...])

        for c_rdma, w_rdma in sends:
            c_rdma.wait_send()
            w_rdma.wait_send()

    f32 = jnp.float32
    return pl.pallas_call(
        body,
        out_shape=jax.ShapeDtypeStruct((B, S, D), f32),
        in_specs=[pl.BlockSpec(memory_space=pltpu.VMEM)] * 8,
        out_specs=pl.BlockSpec(memory_space=pltpu.VMEM),
        scratch_shapes=[
            pltpu.VMEM((BS, DC), f32),
            pltpu.VMEM((DC, 2 * D), f32),
            pltpu.VMEM((N_DEV - 1, BS, DC), f32),
            pltpu.VMEM((N_DEV - 1, DC, 2 * D), f32),
            pltpu.SemaphoreType.DMA((N_DEV - 1,)),
            pltpu.SemaphoreType.DMA((N_DEV - 1,)),
            pltpu.SemaphoreType.DMA((N_DEV - 1,)),
            pltpu.SemaphoreType.DMA((N_DEV - 1,)),
        ],
        compiler_params=pltpu.CompilerParams(collective_id=0),
    )(x, Wdkv, Wuk, Wuv, Wq, Wqr, Wkr, Wo)


# device time: 65529 ns/iter; 1.7403x vs baseline; 1.3893x over previous
import jax
import jax.numpy as jnp
from jax import lax
from jax.experimental import pallas as pl
from jax.experimental.pallas import tpu as pltpu

N_DEV = 8
B, S, H, Dh, Dr = 2, 256, 16, 64, 32
D = 1024
DC = 64
BS = B * S


def _dot(a, b):
    return jnp.dot(a, b, preferred_element_type=jnp.float32)


def kernel(x, Wdkv, Wuk, Wuv, Wq, Wqr, Wkr, Wo):
    def body(x_ref, wdkv_ref, wuk_ref, wuv_ref, wq_ref, wqr_ref, wkr_ref,
             wo_ref, out_ref,
             c_mine, w_mine, c_buf, w_buf,
             c_send_sems, c_recv_sems, w_send_sems, w_recv_sems):
        my = lax.axis_index("i")

        barrier = pltpu.get_barrier_semaphore()
        for k in range(1, N_DEV):
            pl.semaphore_signal(barrier, inc=1, device_id=((my + k) % N_DEV,),
                                device_id_type=pl.DeviceIdType.MESH)
        pl.semaphore_wait(barrier, N_DEV - 1)

        bf16 = jnp.bfloat16
        for b in range(B):
            c_mine[b * S:(b + 1) * S, :] = _dot(
                x_ref[b], wdkv_ref[...]).astype(bf16)
        w_mine[:, :D] = wuk_ref[...].astype(bf16)
        w_mine[:, D:] = wuv_ref[...].astype(bf16)

        sends = []
        for k in range(1, N_DEV):
            peer = (my + k) % N_DEV
            slot = N_DEV - 1 - k
            c_rdma = pltpu.make_async_remote_copy(
                src_ref=c_mine, dst_ref=c_buf.at[slot],
                send_sem=c_send_sems.at[k - 1], recv_sem=c_recv_sems.at[slot],
                device_id=(peer,), device_id_type=pl.DeviceIdType.MESH)
            w_rdma = pltpu.make_async_remote_copy(
                src_ref=w_mine, dst_ref=w_buf.at[slot],
                send_sem=w_send_sems.at[k - 1], recv_sem=w_recv_sems.at[slot],
                device_id=(peer,), device_id_type=pl.DeviceIdType.MESH)
            c_rdma.start()
            w_rdma.start()
            sends.append((c_rdma, w_rdma))

        K = _dot(c_mine[...], w_mine[:, :D])
        V = _dot(c_mine[...], w_mine[:, D:])
        Qs, Qrs, Krs = [], [], []
        for b in range(B):
            xb = x_ref[b]
            Qs.append(_dot(xb, wq_ref[...]))
            Qrs.append(_dot(xb, wqr_ref[...]))
            Krs.append(_dot(xb, wkr_ref[...]))

        for j in range(N_DEV - 1):
            recv_c = pltpu.make_async_remote_copy(
                src_ref=c_buf.at[j], dst_ref=c_buf.at[j],
                send_sem=c_send_sems.at[j], recv_sem=c_recv_sems.at[j],
                device_id=(my,), device_id_type=pl.DeviceIdType.MESH)
            recv_w = pltpu.make_async_remote_copy(
                src_ref=w_buf.at[j], dst_ref=w_buf.at[j],
                send_sem=w_send_sems.at[j], recv_sem=w_recv_sems.at[j],
                device_id=(my,), device_id_type=pl.DeviceIdType.MESH)
            recv_c.wait_recv()
            recv_w.wait_recv()
            K = K + _dot(c_buf[j], w_buf[j, :, :D])
            V = V + _dot(c_buf[j], w_buf[j, :, D:])

        scale = (Dh + Dr) ** -0.5
        for b in range(B):
            Qb, Qrb, Krb = Qs[b], Qrs[b], Krs[b]
            Kb = K[b * S:(b + 1) * S, :]
            Vb = V[b * S:(b + 1) * S, :]
            o_cols = []
            for h in range(H):
                Qh = Qb[:, h * Dh:(h + 1) * Dh]
                Kh = Kb[:, h * Dh:(h + 1) * Dh]
                Qrh = Qrb[:, h * Dr:(h + 1) * Dr]
                s = (_dot(Qh, Kh.T) + _dot(Qrh, Krb.T)) * scale
                m = jnp.max(s, axis=-1, keepdims=True)
                e = jnp.exp(s - m)
                p = e / jnp.sum(e, axis=-1, keepdims=True)
                o_cols.append(_dot(p, Vb[:, h * Dh:(h + 1) * Dh]))
            Ob = jnp.concatenate(o_cols, axis=1)
            out_ref[b, :, :] = _dot(Ob, wo_ref[...])

        for c_rdma, w_rdma in sends:
            c_rdma.wait_send()
            w_rdma.wait_send()

    f32 = jnp.float32
    bf16 = jnp.bfloat16
    return pl.pallas_call(
        body,
        out_shape=jax.ShapeDtypeStruct((B, S, D), f32),
        in_specs=[pl.BlockSpec(memory_space=pltpu.VMEM)] * 8,
        out_specs=pl.BlockSpec(memory_space=pltpu.VMEM),
        scratch_shapes=[
            pltpu.VMEM((BS, DC), bf16),
            pltpu.VMEM((DC, 2 * D), bf16),
            pltpu.VMEM((N_DEV - 1, BS, DC), bf16),
            pltpu.VMEM((N_DEV - 1, DC, 2 * D), bf16),
            pltpu.SemaphoreType.DMA((N_DEV - 1,)),
            pltpu.SemaphoreType.DMA((N_DEV - 1,)),
            pltpu.SemaphoreType.DMA((N_DEV - 1,)),
            pltpu.SemaphoreType.DMA((N_DEV - 1,)),
        ],
        compiler_params=pltpu.CompilerParams(collective_id=0),
    )(x, Wdkv, Wuk, Wuv, Wq, Wqr, Wkr, Wo)


# device time: 41263 ns/iter; 2.7637x vs baseline; 1.5881x over previous
import jax
import jax.numpy as jnp
from jax import lax
from jax.experimental import pallas as pl
from jax.experimental.pallas import tpu as pltpu

N_DEV = 8
B, S, H, Dh, Dr = 2, 256, 16, 64, 32
D = 1024
DC = 64
BS = B * S
HL = H // N_DEV
CK = HL * Dh
CR = HL * Dr


def _dot(a, b):
    return jnp.dot(a, b, preferred_element_type=jnp.float32)


def kernel(x, Wdkv, Wuk, Wuv, Wq, Wqr, Wkr, Wo):
    my_out = lax.axis_index("i")
    wq_my = lax.dynamic_slice(Wq, (0, my_out * CK), (D, CK))
    wqr_my = lax.dynamic_slice(Wqr, (0, my_out * CR), (D, CR))
    wuk_my = lax.dynamic_slice(Wuk, (0, my_out * CK), (DC, CK))
    wuv_my = lax.dynamic_slice(Wuv, (0, my_out * CK), (DC, CK))

    def body(x_ref, wdkv_ref, wuk_ref, wuv_ref, wq_ref, wqr_ref, wkr_ref,
             wo_ref, wq_my_ref, wqr_my_ref, wuk_my_ref, wuv_my_ref, out_ref,
             c_mine, w_send, o_mine, c_buf, w_buf, o_buf,
             c_sems, w_sems, o_sems):
        my = lax.axis_index("i")
        bf16 = jnp.bfloat16

        barrier = pltpu.get_barrier_semaphore()
        for k in range(1, N_DEV):
            pl.semaphore_signal(barrier, inc=1, device_id=((my + k) % N_DEV,),
                                device_id_type=pl.DeviceIdType.MESH)
        pl.semaphore_wait(barrier, N_DEV - 1)

        for b in range(B):
            c_mine[b * S:(b + 1) * S, :] = _dot(
                x_ref[b], wdkv_ref[...]).astype(bf16)
        wuk = wuk_ref[...]
        wuv = wuv_ref[...]
        for d in range(N_DEV):
            w_send[d, :, :CK] = wuk[:, d * CK:(d + 1) * CK].astype(bf16)
            w_send[d, :, CK:] = wuv[:, d * CK:(d + 1) * CK].astype(bf16)

        sends = []
        for k in range(1, N_DEV):
            peer = (my + k) % N_DEV
            slot = N_DEV - 1 - k
            c_rdma = pltpu.make_async_remote_copy(
                src_ref=c_mine, dst_ref=c_buf.at[slot],
                send_sem=c_sems.at[0, k - 1], recv_sem=c_sems.at[1, slot],
                device_id=(peer,), device_id_type=pl.DeviceIdType.MESH)
            w_rdma = pltpu.make_async_remote_copy(
                src_ref=w_send.at[peer], dst_ref=w_buf.at[slot],
                send_sem=w_sems.at[0, k - 1], recv_sem=w_sems.at[1, slot],
                device_id=(peer,), device_id_type=pl.DeviceIdType.MESH)
            c_rdma.start()
            w_rdma.start()
            sends.append((c_rdma, w_rdma))

        Qs, Qrs, Krs = [], [], []
        for b in range(B):
            xb = x_ref[b]
            Qs.append(_dot(xb, wq_my_ref[...]))
            Qrs.append(_dot(xb, wqr_my_ref[...]))
            Krs.append(_dot(xb, wkr_ref[...]))

        w_own_k = wuk_my_ref[...]
        w_own_v = wuv_my_ref[...]
        c_own = c_mine[...]
        K = _dot(c_own, w_own_k.astype(bf16))
        V = _dot(c_own, w_own_v.astype(bf16))
        for j in range(N_DEV - 1):
            recv_c = pltpu.make_async_remote_copy(
                src_ref=c_buf.at[j], dst_ref=c_buf.at[j],
                send_sem=c_sems.at[0, j], recv_sem=c_sems.at[1, j],
                device_id=(my,), device_id_type=pl.DeviceIdType.MESH)
            recv_w = pltpu.make_async_remote_copy(
                src_ref=w_buf.at[j], dst_ref=w_buf.at[j],
                send_sem=w_sems.at[0, j], recv_sem=w_sems.at[1, j],
                device_id=(my,), device_id_type=pl.DeviceIdType.MESH)
            recv_c.wait_recv()
            recv_w.wait_recv()
            K = K + _dot(c_buf[j], w_buf[j, :, :CK])
            V = V + _dot(c_buf[j], w_buf[j, :, CK:])

        scale = (Dh + Dr) ** -0.5
        for b in range(B):
            Qb, Qrb, Krb = Qs[b], Qrs[b], Krs[b]
            Kb = K[b * S:(b + 1) * S, :]
            Vb = V[b * S:(b + 1) * S, :]
            for h in range(HL):
                Qh = Qb[:, h * Dh:(h + 1) * Dh]
                Kh = Kb[:, h * Dh:(h + 1) * Dh]
                Qrh = Qrb[:, h * Dr:(h + 1) * Dr]
                s = (_dot(Qh, Kh.T) + _dot(Qrh, Krb.T)) * scale
                m = jnp.max(s, axis=-1, keepdims=True)
                e = jnp.exp(s - m)
                p = e / jnp.sum(e, axis=-1, keepdims=True)
                o_mine[b * S:(b + 1) * S, h * Dh:(h + 1) * Dh] = _dot(
                    p, Vb[:, h * Dh:(h + 1) * Dh]).astype(bf16)

        o_sends = []
        for k in range(1, N_DEV):
            peer = (my + k) % N_DEV
            slot = N_DEV - 1 - k
            o_rdma = pltpu.make_async_remote_copy(
                src_ref=o_mine, dst_ref=o_buf.at[slot],
                send_sem=o_sems.at[0, k - 1], recv_sem=o_sems.at[1, slot],
                device_id=(peer,), device_id_type=pl.DeviceIdType.MESH)
            o_rdma.start()
            o_sends.append(o_rdma)

        wo_my = wo_ref[pl.ds(my * CK, CK), :]
        out = _dot(o_mine[...], wo_my.astype(bf16))
        for j in range(N_DEV - 1):
            recv_o = pltpu.make_async_remote_copy(
                src_ref=o_buf.at[j], dst_ref=o_buf.at[j],
                send_sem=o_sems.at[0, j], recv_sem=o_sems.at[1, j],
                device_id=(my,), device_id_type=pl.DeviceIdType.MESH)
            recv_o.wait_recv()
            src = (my + j + 1) % N_DEV
            wo_s = wo_ref[pl.ds(src * CK, CK), :]
            out = out + _dot(o_buf[j], wo_s.astype(bf16))
        for b in range(B):
            out_ref[b, :, :] = out[b * S:(b + 1) * S, :]

        for c_rdma, w_rdma in sends:
            c_rdma.wait_send()
            w_rdma.wait_send()
        for o_rdma in o_sends:
            o_rdma.wait_send()

    f32 = jnp.float32
    bf16 = jnp.bfloat16
    return pl.pallas_call(
        body,
        out_shape=jax.ShapeDtypeStruct((B, S, D), f32),
        in_specs=[pl.BlockSpec(memory_space=pltpu.VMEM)] * 12,
        out_specs=pl.BlockSpec(memory_space=pltpu.VMEM),
        scratch_shapes=[
            pltpu.VMEM((BS, DC), bf16),
            pltpu.VMEM((N_DEV, DC, 2 * CK), bf16),
            pltpu.VMEM((BS, CK), bf16),
            pltpu.VMEM((N_DEV - 1, BS, DC), bf16),
            pltpu.VMEM((N_DEV - 1, DC, 2 * CK), bf16),
            pltpu.VMEM((N_DEV - 1, BS, CK), bf16),
            pltpu.SemaphoreType.DMA((2, N_DEV - 1)),
            pltpu.SemaphoreType.DMA((2, N_DEV - 1)),
            pltpu.SemaphoreType.DMA((2, N_DEV - 1)),
        ],
        compiler_params=pltpu.CompilerParams(collective_id=0),
    )(x, Wdkv, Wuk, Wuv, Wq, Wqr, Wkr, Wo, wq_my, wqr_my, wuk_my, wuv_my)


# device time: 34520 ns/iter; 3.3036x vs baseline; 1.1953x over previous
import jax
import jax.numpy as jnp
from jax import lax
from jax.experimental import pallas as pl
from jax.experimental.pallas import tpu as pltpu

N_DEV = 8
B, S, H, Dh, Dr = 2, 256, 16, 64, 32
D = 1024
DC = 64
BS = B * S
HL = H // N_DEV
CK = HL * Dh
CR = HL * Dr


def _dot(a, b):
    return jnp.dot(a, b, preferred_element_type=jnp.float32)


def kernel(x, Wdkv, Wuk, Wuv, Wq, Wqr, Wkr, Wo):
    my_out = lax.axis_index("i")
    wq_my = lax.dynamic_slice(Wq, (0, my_out * CK), (D, CK))
    wqr_my = lax.dynamic_slice(Wqr, (0, my_out * CR), (D, CR))
    wuk_my = lax.dynamic_slice(Wuk, (0, my_out * CK), (DC, CK))
    wuv_my = lax.dynamic_slice(Wuv, (0, my_out * CK), (DC, CK))

    def body(x_ref, wdkv_ref, wuk_ref, wuv_ref, wkr_ref,
             wo_ref, wq_my_ref, wqr_my_ref, wuk_my_ref, wuv_my_ref, out_ref,
             c_mine, w_send, o_mine, c_buf, w_buf, o_buf,
             c_sems, w_sems, o_sems):
        my = lax.axis_index("i")
        bf16 = jnp.bfloat16

        barrier = pltpu.get_barrier_semaphore()
        for k in range(1, N_DEV):
            pl.semaphore_signal(barrier, inc=1, device_id=((my + k) % N_DEV,),
                                device_id_type=pl.DeviceIdType.MESH)
        pl.semaphore_wait(barrier, N_DEV - 1)

        for b in range(B):
            c_mine[b * S:(b + 1) * S, :] = _dot(
                x_ref[b], wdkv_ref[...]).astype(bf16)
        wuk = wuk_ref[...]
        wuv = wuv_ref[...]
        for d in range(N_DEV):
            w_send[d, :, :CK] = wuk[:, d * CK:(d + 1) * CK].astype(bf16)
            w_send[d, :, CK:] = wuv[:, d * CK:(d + 1) * CK].astype(bf16)

        sends = []
        for k in range(1, N_DEV):
            peer = (my + k) % N_DEV
            slot = N_DEV - 1 - k
            c_rdma = pltpu.make_async_remote_copy(
                src_ref=c_mine, dst_ref=c_buf.at[slot],
                send_sem=c_sems.at[0, k - 1], recv_sem=c_sems.at[1, slot],
                device_id=(peer,), device_id_type=pl.DeviceIdType.MESH)
            w_rdma = pltpu.make_async_remote_copy(
                src_ref=w_send.at[peer], dst_ref=w_buf.at[slot],
                send_sem=w_sems.at[0, k - 1], recv_sem=w_sems.at[1, slot],
                device_id=(peer,), device_id_type=pl.DeviceIdType.MESH)
            c_rdma.start()
            w_rdma.start()
            sends.append((c_rdma, w_rdma))

        Qs, Qrs, Krs = [], [], []
        for b in range(B):
            xb = x_ref[b]
            Qs.append(_dot(xb, wq_my_ref[...]))
            Qrs.append(_dot(xb, wqr_my_ref[...]))
            Krs.append(_dot(xb, wkr_ref[...]))

        w_own_k = wuk_my_ref[...]
        w_own_v = wuv_my_ref[...]
        c_own = c_mine[...]
        K = _dot(c_own, w_own_k.astype(bf16))
        V = _dot(c_own, w_own_v.astype(bf16))
        for j in range(N_DEV - 1):
            recv_c = pltpu.make_async_remote_copy(
                src_ref=c_buf.at[j], dst_ref=c_buf.at[j],
                send_sem=c_sems.at[0, j], recv_sem=c_sems.at[1, j],
                device_id=(my,), device_id_type=pl.DeviceIdType.MESH)
            recv_w = pltpu.make_async_remote_copy(
                src_ref=w_buf.at[j], dst_ref=w_buf.at[j],
                send_sem=w_sems.at[0, j], recv_sem=w_sems.at[1, j],
                device_id=(my,), device_id_type=pl.DeviceIdType.MESH)
            recv_c.wait_recv()
            recv_w.wait_recv()
            K = K + _dot(c_buf[j], w_buf[j, :, :CK])
            V = V + _dot(c_buf[j], w_buf[j, :, CK:])

        scale = (Dh + Dr) ** -0.5
        o_sends = []
        for b in range(B):
            Qb, Qrb, Krb = Qs[b], Qrs[b], Krs[b]
            Kb = K[b * S:(b + 1) * S, :]
            Vb = V[b * S:(b + 1) * S, :]
            for h in range(HL):
                Qh = Qb[:, h * Dh:(h + 1) * Dh]
                Kh = Kb[:, h * Dh:(h + 1) * Dh]
                Qrh = Qrb[:, h * Dr:(h + 1) * Dr]
                s = (_dot(Qh, Kh.T) + _dot(Qrh, Krb.T)) * scale
                m = jnp.max(s, axis=-1, keepdims=True)
                e = jnp.exp(s - m)
                p = e / jnp.sum(e, axis=-1, keepdims=True)
                o_mine[b * S:(b + 1) * S, h * Dh:(h + 1) * Dh] = _dot(
                    p, Vb[:, h * Dh:(h + 1) * Dh]).astype(bf16)
            for k in range(1, N_DEV):
                peer = (my + k) % N_DEV
                slot = N_DEV - 1 - k
                o_rdma = pltpu.make_async_remote_copy(
                    src_ref=o_mine.at[b * S:(b + 1) * S, :],
                    dst_ref=o_buf.at[slot, b * S:(b + 1) * S, :],
                    send_sem=o_sems.at[b, 0, k - 1],
                    recv_sem=o_sems.at[b, 1, slot],
                    device_id=(peer,), device_id_type=pl.DeviceIdType.MESH)
                o_rdma.start()
                o_sends.append(o_rdma)

        wo_my = wo_ref[pl.ds(my * CK, CK), :]
        out = _dot(o_mine[...], wo_my.astype(bf16))
        for j in range(N_DEV - 1):
            for b in range(B):
                recv_o = pltpu.make_async_remote_copy(
                    src_ref=o_buf.at[j, b * S:(b + 1) * S, :],
                    dst_ref=o_buf.at[j, b * S:(b + 1) * S, :],
                    send_sem=o_sems.at[b, 0, j], recv_sem=o_sems.at[b, 1, j],
                    device_id=(my,), device_id_type=pl.DeviceIdType.MESH)
                recv_o.wait_recv()
            src = (my + j + 1) % N_DEV
            wo_s = wo_ref[pl.ds(src * CK, CK), :]
            out = out + _dot(o_buf[j], wo_s.astype(bf16))
        for b in range(B):
            out_ref[b, :, :] = out[b * S:(b + 1) * S, :]

        for c_rdma, w_rdma in sends:
            c_rdma.wait_send()
            w_rdma.wait_send()
        for o_rdma in o_sends:
            o_rdma.wait_send()

    f32 = jnp.float32
    bf16 = jnp.bfloat16
    return pl.pallas_call(
        body,
        out_shape=jax.ShapeDtypeStruct((B, S, D), f32),
        in_specs=[pl.BlockSpec(memory_space=pltpu.VMEM)] * 10,
        out_specs=pl.BlockSpec(memory_space=pltpu.VMEM),
        scratch_shapes=[
            pltpu.VMEM((BS, DC), bf16),
            pltpu.VMEM((N_DEV, DC, 2 * CK), bf16),
            pltpu.VMEM((BS, CK), bf16),
            pltpu.VMEM((N_DEV - 1, BS, DC), bf16),
            pltpu.VMEM((N_DEV - 1, DC, 2 * CK), bf16),
            pltpu.VMEM((N_DEV - 1, BS, CK), bf16),
            pltpu.SemaphoreType.DMA((2, N_DEV - 1)),
            pltpu.SemaphoreType.DMA((2, N_DEV - 1)),
            pltpu.SemaphoreType.DMA((B, 2, N_DEV - 1)),
        ],
        compiler_params=pltpu.CompilerParams(collective_id=0),
    )(x, Wdkv, Wuk, Wuv, Wkr, Wo, wq_my, wqr_my, wuk_my, wuv_my)


# device time: 34513 ns/iter; 3.3043x vs baseline; 1.0002x over previous
import jax
import jax.numpy as jnp
from jax import lax
from jax.experimental import pallas as pl
from jax.experimental.pallas import tpu as pltpu

N_DEV = 8
B, S, H, Dh, Dr = 2, 256, 16, 64, 32
D = 1024
DC = 64
BS = B * S
HL = H // N_DEV
CK = HL * Dh
CR = HL * Dr


def _dot(a, b):
    return jnp.dot(a, b, preferred_element_type=jnp.float32)


def kernel(x, Wdkv, Wuk, Wuv, Wq, Wqr, Wkr, Wo):
    my_out = lax.axis_index("i")
    wq_my = lax.dynamic_slice(Wq, (0, my_out * CK), (D, CK))
    wqr_my = lax.dynamic_slice(Wqr, (0, my_out * CR), (D, CR))
    wuk_my = lax.dynamic_slice(Wuk, (0, my_out * CK), (DC, CK))
    wuv_my = lax.dynamic_slice(Wuv, (0, my_out * CK), (DC, CK))

    def body(x_ref, wdkv_ref, wuk_ref, wuv_ref, wkr_ref,
             wo_ref, wq_my_ref, wqr_my_ref, wuk_my_ref, wuv_my_ref, out_ref,
             c_mine, w_send, o_mine, c_buf, w_buf, o_buf, wo_v, out_v,
             c_sems, w_sems, o_sems, wo_sem, out_sem):
        my = lax.axis_index("i")
        bf16 = jnp.bfloat16

        wo_copy = pltpu.make_async_copy(wo_ref, wo_v, wo_sem)
        wo_copy.start()

        barrier = pltpu.get_barrier_semaphore()
        for k in range(1, N_DEV):
            pl.semaphore_signal(barrier, inc=1, device_id=((my + k) % N_DEV,),
                                device_id_type=pl.DeviceIdType.MESH)

        for b in range(B):
            c_mine[b * S:(b + 1) * S, :] = _dot(
                x_ref[b], wdkv_ref[...]).astype(bf16)
        wuk = wuk_ref[...]
        wuv = wuv_ref[...]
        for d in range(N_DEV):
            w_send[d, :, :CK] = wuk[:, d * CK:(d + 1) * CK].astype(bf16)
            w_send[d, :, CK:] = wuv[:, d * CK:(d + 1) * CK].astype(bf16)

        pl.semaphore_wait(barrier, N_DEV - 1)

        sends = []
        for k in range(1, N_DEV):
            peer = (my + k) % N_DEV
            slot = N_DEV - 1 - k
            c_rdma = pltpu.make_async_remote_copy(
                src_ref=c_mine, dst_ref=c_buf.at[slot],
                send_sem=c_sems.at[0, k - 1], recv_sem=c_sems.at[1, slot],
                device_id=(peer,), device_id_type=pl.DeviceIdType.MESH)
            w_rdma = pltpu.make_async_remote_copy(
                src_ref=w_send.at[peer], dst_ref=w_buf.at[slot],
                send_sem=w_sems.at[0, k - 1], recv_sem=w_sems.at[1, slot],
                device_id=(peer,), device_id_type=pl.DeviceIdType.MESH)
            c_rdma.start()
            w_rdma.start()
            sends.append((c_rdma, w_rdma))

        Qs, Qrs, Krs = [], [], []
        for b in range(B):
            xb = x_ref[b]
            Qs.append(_dot(xb, wq_my_ref[...]))
            Qrs.append(_dot(xb, wqr_my_ref[...]))
            Krs.append(_dot(xb, wkr_ref[...]))

        w_own_k = wuk_my_ref[...]
        w_own_v = wuv_my_ref[...]
        c_own = c_mine[...]
        K = _dot(c_own, w_own_k.astype(bf16))
        V = _dot(c_own, w_own_v.astype(bf16))
        for j in range(N_DEV - 1):
            recv_c = pltpu.make_async_remote_copy(
                src_ref=c_buf.at[j], dst_ref=c_buf.at[j],
                send_sem=c_sems.at[0, j], recv_sem=c_sems.at[1, j],
                device_id=(my,), device_id_type=pl.DeviceIdType.MESH)
            recv_w = pltpu.make_async_remote_copy(
                src_ref=w_buf.at[j], dst_ref=w_buf.at[j],
                send_sem=w_sems.at[0, j], recv_sem=w_sems.at[1, j],
                device_id=(my,), device_id_type=pl.DeviceIdType.MESH)
            recv_c.wait_recv()
            recv_w.wait_recv()
            K = K + _dot(c_buf[j], w_buf[j, :, :CK])
            V = V + _dot(c_buf[j], w_buf[j, :, CK:])

        scale = (Dh + Dr) ** -0.5
        o_sends = []
        for b in range(B):
            Qb, Qrb, Krb = Qs[b], Qrs[b], Krs[b]
            Kb = K[b * S:(b + 1) * S, :]
            Vb = V[b * S:(b + 1) * S, :]
            for h in range(HL):
                Qh = Qb[:, h * Dh:(h + 1) * Dh]
                Kh = Kb[:, h * Dh:(h + 1) * Dh]
                Qrh = Qrb[:, h * Dr:(h + 1) * Dr]
                s = (_dot(Qh, Kh.T) + _dot(Qrh, Krb.T)) * scale
                m = jnp.max(s, axis=-1, keepdims=True)
                e = jnp.exp(s - m)
                p = e / jnp.sum(e, axis=-1, keepdims=True)
                o_mine[b * S:(b + 1) * S, h * Dh:(h + 1) * Dh] = _dot(
                    p, Vb[:, h * Dh:(h + 1) * Dh]).astype(bf16)
            for k in range(1, N_DEV):
                peer = (my + k) % N_DEV
                slot = N_DEV - 1 - k
                o_rdma = pltpu.make_async_remote_copy(
                    src_ref=o_mine.at[b * S:(b + 1) * S, :],
                    dst_ref=o_buf.at[slot, b * S:(b + 1) * S, :],
                    send_sem=o_sems.at[b, 0, k - 1],
                    recv_sem=o_sems.at[b, 1, slot],
                    device_id=(peer,), device_id_type=pl.DeviceIdType.MESH)
                o_rdma.start()
                o_sends.append(o_rdma)

        wo_copy.wait()
        wo_my = wo_v[pl.ds(my * CK, CK), :]
        out = _dot(o_mine[...], wo_my.astype(bf16))
        for j in range(N_DEV - 1):
            for b in range(B):
                recv_o = pltpu.make_async_remote_copy(
                    src_ref=o_buf.at[j, b * S:(b + 1) * S, :],
                    dst_ref=o_buf.at[j, b * S:(b + 1) * S, :],
                    send_sem=o_sems.at[b, 0, j], recv_sem=o_sems.at[b, 1, j],
                    device_id=(my,), device_id_type=pl.DeviceIdType.MESH)
                recv_o.wait_recv()
            src = (my + j + 1) % N_DEV
            wo_s = wo_v[pl.ds(src * CK, CK), :]
            out = out + _dot(o_buf[j], wo_s.astype(bf16))
        for b in range(B):
            out_v[b, :, :] = out[b * S:(b + 1) * S, :]
        out_copy = pltpu.make_async_copy(out_v, out_ref, out_sem)
        out_copy.start()

        for c_rdma, w_rdma in sends:
            c_rdma.wait_send()
            w_rdma.wait_send()
        for o_rdma in o_sends:
            o_rdma.wait_send()
        out_copy.wait()

    f32 = jnp.float32
    bf16 = jnp.bfloat16
    return pl.pallas_call(
        body,
        out_shape=jax.ShapeDtypeStruct((B, S, D), f32),
        in_specs=(
            [pl.BlockSpec(memory_space=pltpu.VMEM)] * 5
            + [pl.BlockSpec(memory_space=pl.ANY)]
            + [pl.BlockSpec(memory_space=pltpu.VMEM)] * 4
        ),
        out_specs=pl.BlockSpec(memory_space=pl.ANY),
        scratch_shapes=[
            pltpu.VMEM((BS, DC), bf16),
            pltpu.VMEM((N_DEV, DC, 2 * CK), bf16),
            pltpu.VMEM((BS, CK), bf16),
            pltpu.VMEM((N_DEV - 1, BS, DC), bf16),
            pltpu.VMEM((N_DEV - 1, DC, 2 * CK), bf16),
            pltpu.VMEM((N_DEV - 1, BS, CK), bf16),
            pltpu.VMEM((D, D), f32),
            pltpu.VMEM((B, S, D), f32),
            pltpu.SemaphoreType.DMA((2, N_DEV - 1)),
            pltpu.SemaphoreType.DMA((2, N_DEV - 1)),
            pltpu.SemaphoreType.DMA((B, 2, N_DEV - 1)),
            pltpu.SemaphoreType.DMA,
            pltpu.SemaphoreType.DMA,
        ],
        compiler_params=pltpu.CompilerParams(collective_id=0),
    )(x, Wdkv, Wuk, Wuv, Wkr, Wo, wq_my, wqr_my, wuk_my, wuv_my)
